# baseline (speedup 1.0000x reference)
import jax
import jax.numpy as jnp
from jax import lax
from jax.experimental import pallas as pl
from jax.experimental.pallas import tpu as pltpu
from jax.experimental.pallas import tpu_sc as plsc

VOCAB = 100000
MAX_LEN = 200
EMB = 128
BATCH = 4096
SEQ = 200

NUM_WORKERS = 32
TOK_PER_W = BATCH * SEQ // NUM_WORKERS
CHUNK = 128
N = TOK_PER_W // CHUNK
NBUF = 5
NVAR = 25
SKEW = 3


def _pos_plan(v):
    s = (CHUNK * v) % MAX_LEN
    if s + CHUNK <= MAX_LEN:
        return [(s, 0, CHUNK)]
    n1 = MAX_LEN - s
    return [(s, 0, n1), (0, n1, CHUNK - n1)]


_mesh = plsc.VectorSubcoreMesh(core_axis_name="c", subcore_axis_name="s")

_scratch = (
    [pltpu.VMEM_SHARED((MAX_LEN, EMB), jnp.float32)]
    + [pltpu.VMEM((N, CHUNK), jnp.int32)]
    + [pltpu.VMEM((CHUNK, EMB), jnp.float32) for _ in range(NBUF)]
    + [pltpu.SemaphoreType.DMA for _ in range(3 * NBUF)]
)


@jax.jit
def _embed_call(x2d, wt, pos):
    @pl.kernel(
        out_type=jax.ShapeDtypeStruct((BATCH * SEQ, EMB), jnp.float32),
        mesh=_mesh,
        scratch_types=_scratch,
    )
    def _embed(x_hbm, wt_hbm, pos_hbm, out_hbm, pos_sh, idx2d, *scr):
        bufs = scr[0:NBUF]
        sem_init = scr[NBUF:2 * NBUF]
        sem_g = scr[2 * NBUF:3 * NBUF]
        sem_wb = scr[3 * NBUF:4 * NBUF]

        sid = lax.axis_index("s")
        wid = sid * 2 + lax.axis_index("c")
        base = wid * TOK_PER_W

        @pl.when(sid == 0)
        def _load_pos():
            pltpu.sync_copy(pos_hbm, pos_sh)

        plsc.subcore_barrier()

        pltpu.sync_copy(x_hbm.at[pl.ds(wid * N, N)], idx2d)

        def init_descs(v, b):
            src = pos_sh if v % 2 == 0 else pos_hbm
            return [
                pltpu.make_async_copy(
                    src.at[pl.ds(po, n)],
                    bufs[b].at[pl.ds(bo, n)],
                    sem_init[b])
                for po, bo, n in _pos_plan(v)
            ]

        def d_gat(g, b):
            return pltpu.make_async_copy(
                wt_hbm.at[idx2d.at[g]], bufs[b], sem_g[b])

        def d_wb(g, b):
            return pltpu.make_async_copy(
                bufs[b], out_hbm.at[pl.ds(base + g * CHUNK, CHUNK)],
                sem_wb[b])

        def issue_pre(v, b):
            for d in init_descs(v, b):
                d.start()

        def issue_gather(g, v, b):
            for d in init_descs(v, b):
                d.wait()
            d_gat(g, b).start(add=True)

        def issue_wb(g, b):
            d_gat(g, b).wait()
            d_wb(g, b).start()

        def body(i, phase, drain):
            b0 = phase % NBUF
            b3 = (phase + SKEW) % NBUF
            v3 = (phase + SKEW) % NVAR
            if drain:
                d_wb(i - (NBUF - SKEW), b3).wait()
            issue_pre(v3, b3)
            issue_wb(i, b0)
            issue_gather(i + SKEW, v3, b3)

        for g in range(SKEW):
            issue_pre(g, g)
            issue_gather(g, g, g)
        for i in range(NBUF - SKEW):
            body(i, i, drain=False)

        steady_lo = NBUF - SKEW
        steady_hi = steady_lo + ((N - SKEW - steady_lo) // NVAR) * NVAR

        @pl.loop(steady_lo, steady_hi, step=NVAR)
        def _steady(i0):
            for db in range(NVAR):
                body(i0 + db, steady_lo + db, drain=True)

        for i in range(steady_hi, N - SKEW):
            body(i, i, drain=True)

        for g in range(N - SKEW, N):
            issue_wb(g, g % NBUF)
        for g in range(N - NBUF, N):
            d_wb(g, g % NBUF).wait()

    return _embed(x2d, wt, pos)


def kernel(X, word_table, pos_table):
    out = _embed_call(X.reshape(-1, CHUNK), word_table, pos_table)
    return out.reshape(BATCH, SEQ, EMB)


# speedup vs baseline: 2.0827x; 2.0827x over previous
import jax
import jax.numpy as jnp
from jax import lax
from jax.experimental import pallas as pl
from jax.experimental.pallas import tpu as pltpu
from jax.experimental.pallas import tpu_sc as plsc

VOCAB = 100000
MAX_LEN = 200
EMB = 128
BATCH = 4096
SEQ = 200

NUM_WORKERS = 32
TOK_PER_W = BATCH * SEQ // NUM_WORKERS
CHUNK = 128
N = TOK_PER_W // CHUNK
NBUF = 5
NVAR = 25
SKEW = 3


def _pos_plan(v):
    s = (CHUNK * v) % MAX_LEN
    if s + CHUNK <= MAX_LEN:
        return [(s, 0, CHUNK)]
    n1 = MAX_LEN - s
    return [(s, 0, n1), (0, n1, CHUNK - n1)]


_mesh = plsc.VectorSubcoreMesh(core_axis_name="c", subcore_axis_name="s")

_scratch = (
    [pltpu.VMEM_SHARED((MAX_LEN, EMB), jnp.float32)]
    + [pltpu.VMEM((N, CHUNK), jnp.int32)]
    + [pltpu.VMEM((CHUNK, EMB), jnp.float32) for _ in range(NBUF)]
    + [pltpu.SemaphoreType.DMA for _ in range(3 * NBUF)]
)


@jax.jit
def _embed_call(x2d, wt, pos):
    @pl.kernel(
        out_type=jax.ShapeDtypeStruct((BATCH * SEQ, EMB), jnp.float32),
        mesh=_mesh,
        scratch_types=_scratch,
    )
    def _embed(x_hbm, wt_hbm, pos_hbm, out_hbm, pos_sh, idx2d, *scr):
        bufs = scr[0:NBUF]
        sem_init = scr[NBUF:2 * NBUF]
        sem_g = scr[2 * NBUF:3 * NBUF]
        sem_wb = scr[3 * NBUF:4 * NBUF]

        sid = lax.axis_index("s")
        wid = sid * 2 + lax.axis_index("c")
        base = wid * TOK_PER_W

        @pl.when(sid == 0)
        def _load_pos():
            pltpu.sync_copy(pos_hbm, pos_sh)

        plsc.subcore_barrier()

        pltpu.sync_copy(x_hbm.at[pl.ds(wid * N, N)], idx2d)

        def init_descs(v, b):
            return [
                pltpu.make_async_copy(
                    pos_sh.at[pl.ds(po, n)],
                    bufs[b].at[pl.ds(bo, n)],
                    sem_init[b])
                for po, bo, n in _pos_plan(v)
            ]

        HALF = CHUNK // 2

        def d_gat(g, b):
            return [
                pltpu.make_async_copy(
                    wt_hbm.at[idx2d.at[g, pl.ds(h * HALF, HALF)]],
                    bufs[b].at[pl.ds(h * HALF, HALF)],
                    sem_g[b])
                for h in range(2)
            ]

        def d_wb(g, b):
            return pltpu.make_async_copy(
                bufs[b], out_hbm.at[pl.ds(base + g * CHUNK, CHUNK)],
                sem_wb[b])

        def issue_pre(v, b):
            for d in init_descs(v, b):
                d.start()

        def issue_gather(g, v, b):
            for d in init_descs(v, b):
                d.wait()
            for d in d_gat(g, b):
                d.start(add=True)

        def issue_wb(g, b):
            for d in d_gat(g, b):
                d.wait()
            d_wb(g, b).start()

        def body(i, phase, drain):
            b0 = phase % NBUF
            b3 = (phase + SKEW) % NBUF
            v3 = (phase + SKEW) % NVAR
            if drain:
                d_wb(i - (NBUF - SKEW), b3).wait()
            issue_pre(v3, b3)
            issue_wb(i, b0)
            issue_gather(i + SKEW, v3, b3)

        for g in range(SKEW):
            issue_pre(g, g)
            issue_gather(g, g, g)
        for i in range(NBUF - SKEW):
            body(i, i, drain=False)

        steady_lo = NBUF - SKEW
        steady_hi = steady_lo + ((N - SKEW - steady_lo) // NVAR) * NVAR

        @pl.loop(steady_lo, steady_hi, step=NVAR)
        def _steady(i0):
            for db in range(NVAR):
                body(i0 + db, steady_lo + db, drain=True)

        for i in range(steady_hi, N - SKEW):
            body(i, i, drain=True)

        for g in range(N - SKEW, N):
            issue_wb(g, g % NBUF)
        for g in range(N - NBUF, N):
            d_wb(g, g % NBUF).wait()

    return _embed(x2d, wt, pos)


def kernel(X, word_table, pos_table):
    out = _embed_call(X.reshape(-1, CHUNK), word_table, pos_table)
    return out.reshape(BATCH, SEQ, EMB)


# speedup vs baseline: 2.0973x; 1.0070x over previous
import jax
import jax.numpy as jnp
from jax import lax
from jax.experimental import pallas as pl
from jax.experimental.pallas import tpu as pltpu
from jax.experimental.pallas import tpu_sc as plsc

VOCAB = 100000
MAX_LEN = 200
EMB = 128
BATCH = 4096
SEQ = 200

NUM_WORKERS = 32
TOK_PER_W = BATCH * SEQ // NUM_WORKERS
CHUNK = 80
N = TOK_PER_W // CHUNK
NBUF = 5
NVAR = 5
SKEW = 3


def _pos_plan(v):
    s = (CHUNK * v) % MAX_LEN
    if s + CHUNK <= MAX_LEN:
        return [(s, 0, CHUNK)]
    n1 = MAX_LEN - s
    return [(s, 0, n1), (0, n1, CHUNK - n1)]


_mesh = plsc.VectorSubcoreMesh(core_axis_name="c", subcore_axis_name="s")

_scratch = (
    [pltpu.VMEM_SHARED((MAX_LEN, EMB), jnp.float32)]
    + [pltpu.VMEM((N, CHUNK), jnp.int32)]
    + [pltpu.VMEM((CHUNK, EMB), jnp.float32) for _ in range(NBUF)]
    + [pltpu.SemaphoreType.DMA for _ in range(3 * NBUF)]
)


@jax.jit
def _embed_call(x2d, wt, pos):
    @pl.kernel(
        out_type=jax.ShapeDtypeStruct((BATCH * SEQ, EMB), jnp.float32),
        mesh=_mesh,
        scratch_types=_scratch,
    )
    def _embed(x_hbm, wt_hbm, pos_hbm, out_hbm, pos_sh, idx2d, *scr):
        bufs = scr[0:NBUF]
        sem_init = scr[NBUF:2 * NBUF]
        sem_g = scr[2 * NBUF:3 * NBUF]
        sem_wb = scr[3 * NBUF:4 * NBUF]

        sid = lax.axis_index("s")
        wid = sid * 2 + lax.axis_index("c")
        base = wid * TOK_PER_W

        @pl.when(sid == 0)
        def _load_pos():
            pltpu.sync_copy(pos_hbm, pos_sh)

        plsc.subcore_barrier()

        pltpu.sync_copy(x_hbm.at[pl.ds(wid * N, N)], idx2d)

        def init_descs(v, b):
            return [
                pltpu.make_async_copy(
                    pos_sh.at[pl.ds(po, n)],
                    bufs[b].at[pl.ds(bo, n)],
                    sem_init[b])
                for po, bo, n in _pos_plan(v)
            ]

        def d_gat(g, b):
            return [pltpu.make_async_copy(
                wt_hbm.at[idx2d.at[g]], bufs[b], sem_g[b])]

        def d_wb(g, b):
            return pltpu.make_async_copy(
                bufs[b], out_hbm.at[pl.ds(base + g * CHUNK, CHUNK)],
                sem_wb[b])

        def issue_pre(v, b):
            for d in init_descs(v, b):
                d.start()

        def issue_gather(g, v, b):
            for d in init_descs(v, b):
                d.wait()
            for d in d_gat(g, b):
                d.start(add=True)

        def issue_wb(g, b):
            for d in d_gat(g, b):
                d.wait()
            d_wb(g, b).start()

        def body(i, phase, drain):
            b0 = phase % NBUF
            b3 = (phase + SKEW) % NBUF
            v3 = (phase + SKEW) % NVAR
            if drain:
                d_wb(i - (NBUF - SKEW), b3).wait()
            issue_pre(v3, b3)
            issue_wb(i, b0)
            issue_gather(i + SKEW, v3, b3)

        for g in range(SKEW):
            issue_pre(g, g)
            issue_gather(g, g, g)
        for i in range(NBUF - SKEW):
            body(i, i, drain=False)

        steady_lo = NBUF - SKEW
        steady_hi = steady_lo + ((N - SKEW - steady_lo) // NVAR) * NVAR

        @pl.loop(steady_lo, steady_hi, step=NVAR)
        def _steady(i0):
            for db in range(NVAR):
                body(i0 + db, steady_lo + db, drain=True)

        for i in range(steady_hi, N - SKEW):
            body(i, i, drain=True)

        for g in range(N - SKEW, N):
            issue_wb(g, g % NBUF)
        for g in range(N - NBUF, N):
            d_wb(g, g % NBUF).wait()

    return _embed(x2d, wt, pos)


def kernel(X, word_table, pos_table):
    out = _embed_call(X.reshape(-1, CHUNK), word_table, pos_table)
    return out.reshape(BATCH, SEQ, EMB)
